# baseline (speedup 1.0000x reference)
import functools

import jax
import jax.numpy as jnp
from jax import lax
from jax.experimental import pallas as pl
from jax.experimental.pallas import tpu as pltpu
from jax.experimental.pallas import tpu_sc as plsc

V = 100000
D = 300
R = 10
N_RELS = 40
B = 4096

GROUP = 4
VG = V // GROUP
DG = D * GROUP

NC, NS = 2, 16
NW = NC * NS
SIDE_PER_W = B // NW
CHUNK = 64

BLK = 512
NBLK = B // BLK


def _sc_gather(table_g, q_l, q_r):
    mesh = plsc.VectorSubcoreMesh(core_axis_name="c", subcore_axis_name="s")

    @functools.partial(
        pl.kernel,
        mesh=mesh,
        compiler_params=pltpu.CompilerParams(use_tc_tiling_on_sc=False),
        out_type=(
            jax.ShapeDtypeStruct((B, DG), jnp.float32),
            jax.ShapeDtypeStruct((B, DG), jnp.float32),
        ),
        scratch_types=[
            pltpu.VMEM((CHUNK,), jnp.int32),
            pltpu.VMEM((CHUNK, DG), jnp.float32),
            pltpu.SemaphoreType.DMA,
        ],
    )
    def gather_kernel(table_hbm, ql_hbm, qr_hbm, ol_hbm, or_hbm,
                      idx_v, rows_v, sem):
        wid = lax.axis_index("s") * NC + lax.axis_index("c")
        base = wid * SIDE_PER_W
        for i_hbm, o_hbm in ((ql_hbm, ol_hbm), (qr_hbm, or_hbm)):
            for c in range(SIDE_PER_W // CHUNK):
                off = base + c * CHUNK
                pltpu.sync_copy(i_hbm.at[pl.ds(off, CHUNK)], idx_v)
                pltpu.async_copy(table_hbm.at[idx_v], rows_v, sem).wait()
                pltpu.sync_copy(rows_v, o_hbm.at[pl.ds(off, CHUNK)])

    return gather_kernel(table_g, q_l, q_r)


def _tc_body(l4_ref, r4_ref, ml_ref, mr_ref, rel1h_ref, relt_ref, w_ref,
             bb_ref, tm_ref, to_ref, out_ref):
    rel_emb = jnp.dot(rel1h_ref[...], relt_ref[...],
                      preferred_element_type=jnp.float32)
    l_sel = jnp.sum(l4_ref[...] * ml_ref[...], axis=1)
    r_sel = jnp.sum(r4_ref[...] * mr_ref[...], axis=1)
    p = jnp.zeros((BLK, D), jnp.float32)
    w = w_ref[...]
    for k in range(R):
        a = (l_sel * rel_emb[:, k:k + 1]).astype(jnp.bfloat16)
        p = p + jnp.dot(a, w[k], preferred_element_type=jnp.float32)
    energy = (jnp.sum(p * r_sel, axis=1, keepdims=True)
              + jnp.sum(rel_emb * bb_ref[...], axis=1, keepdims=True))
    out_ref[...] = energy * tm_ref[0, 0] + to_ref[0, 0]


def kernel(rels, terms_L, terms_R, term_table, rel_table, bil_w, bil_b,
           truth_multiplier, truth_offset):
    table_g = term_table.reshape(VG, DG)
    q_l = terms_L // GROUP
    q_r = terms_R // GROUP
    gl, gr = _sc_gather(table_g, q_l, q_r)
    l4 = gl.reshape(B, GROUP, D)
    r4 = gr.reshape(B, GROUP, D)

    cgrid = jnp.arange(GROUP, dtype=jnp.int32)[None, :]
    mask_l = (terms_L[:, None] % GROUP == cgrid).astype(jnp.float32)
    mask_l = mask_l.reshape(B, GROUP, 1)
    mask_r = (terms_R[:, None] % GROUP == cgrid).astype(jnp.float32)
    mask_r = mask_r.reshape(B, GROUP, 1)
    rel1h = (rels[:, None] == jnp.arange(N_RELS, dtype=jnp.int32)[None, :])
    rel1h = rel1h.astype(jnp.float32)
    w_bf = bil_w.astype(jnp.bfloat16)
    bb = bil_b.reshape(1, R)
    tm = truth_multiplier.reshape(1, 1)
    to = truth_offset.reshape(1, 1)

    out = pl.pallas_call(
        _tc_body,
        grid=(NBLK,),
        in_specs=[
            pl.BlockSpec((BLK, GROUP, D), lambda i: (i, 0, 0)),
            pl.BlockSpec((BLK, GROUP, D), lambda i: (i, 0, 0)),
            pl.BlockSpec((BLK, GROUP, 1), lambda i: (i, 0, 0)),
            pl.BlockSpec((BLK, GROUP, 1), lambda i: (i, 0, 0)),
            pl.BlockSpec((BLK, N_RELS), lambda i: (i, 0)),
            pl.BlockSpec((N_RELS, R), lambda i: (0, 0)),
            pl.BlockSpec((R, D, D), lambda i: (0, 0, 0)),
            pl.BlockSpec((1, R), lambda i: (0, 0)),
            pl.BlockSpec((1, 1), lambda i: (0, 0)),
            pl.BlockSpec((1, 1), lambda i: (0, 0)),
        ],
        out_specs=pl.BlockSpec((BLK, 1), lambda i: (i, 0)),
        out_shape=jax.ShapeDtypeStruct((B, 1), jnp.float32),
    )(l4, r4, mask_l, mask_r, rel1h, rel_table, w_bf, bb, tm, to)
    return out.reshape(B)


# speedup vs baseline: 1.4155x; 1.4155x over previous
import functools

import jax
import jax.numpy as jnp
from jax import lax
from jax.experimental import pallas as pl
from jax.experimental.pallas import tpu as pltpu
from jax.experimental.pallas import tpu_sc as plsc

V = 100000
D = 300
DP = 384
R = 10
N_RELS = 40
B = 4096

NC, NS = 2, 16
NW = NC * NS
SIDE_PER_W = B // NW

BLK = 512
NBLK = B // BLK


def _sc_gather(table, t_l, t_r):
    mesh = plsc.VectorSubcoreMesh(core_axis_name="c", subcore_axis_name="s")

    @functools.partial(
        pl.kernel,
        mesh=mesh,
        out_type=(
            jax.ShapeDtypeStruct((B, DP), jnp.float32),
            jax.ShapeDtypeStruct((B, DP), jnp.float32),
        ),
        scratch_types=[
            pltpu.VMEM((SIDE_PER_W,), jnp.int32),
            pltpu.VMEM((SIDE_PER_W, DP), jnp.float32),
            pltpu.SemaphoreType.DMA,
        ],
    )
    def gather_kernel(table_hbm, tl_hbm, tr_hbm, ol_hbm, or_hbm,
                      idx_v, rows_v, sem):
        wid = lax.axis_index("s") * NC + lax.axis_index("c")
        base = wid * SIDE_PER_W
        for i_hbm, o_hbm in ((tl_hbm, ol_hbm), (tr_hbm, or_hbm)):
            pltpu.sync_copy(i_hbm.at[pl.ds(base, SIDE_PER_W)], idx_v)
            pltpu.async_copy(table_hbm.at[idx_v], rows_v, sem).wait()
            pltpu.sync_copy(rows_v, o_hbm.at[pl.ds(base, SIDE_PER_W)])

    return gather_kernel(table, t_l, t_r)


def _tc_body(l_ref, r_ref, rel1h_ref, relt_ref, w_ref, bb_ref, tm_ref,
             to_ref, out_ref):
    rel_emb = jnp.dot(rel1h_ref[...], relt_ref[...],
                      preferred_element_type=jnp.float32)
    l_sel = l_ref[...]
    r_sel = r_ref[...]
    p = jnp.zeros((BLK, DP), jnp.float32)
    w = w_ref[...]
    for k in range(R):
        a = (l_sel * rel_emb[:, k:k + 1]).astype(jnp.bfloat16)
        p = p + jnp.dot(a, w[k], preferred_element_type=jnp.float32)
    energy = (jnp.sum(p * r_sel, axis=1, keepdims=True)
              + jnp.sum(rel_emb * bb_ref[...], axis=1, keepdims=True))
    out_ref[...] = energy * tm_ref[0, 0] + to_ref[0, 0]


def kernel(rels, terms_L, terms_R, term_table, rel_table, bil_w, bil_b,
           truth_multiplier, truth_offset):
    tpad = jnp.pad(term_table, ((0, 0), (0, DP - D)))
    gl, gr = _sc_gather(tpad, terms_L, terms_R)

    rel1h = (rels[:, None] == jnp.arange(N_RELS, dtype=jnp.int32)[None, :])
    rel1h = rel1h.astype(jnp.float32)
    w_bf = jnp.pad(bil_w, ((0, 0), (0, DP - D), (0, DP - D))).astype(jnp.bfloat16)
    bb = bil_b.reshape(1, R)
    tm = truth_multiplier.reshape(1, 1)
    to = truth_offset.reshape(1, 1)

    out = pl.pallas_call(
        _tc_body,
        grid=(NBLK,),
        in_specs=[
            pl.BlockSpec((BLK, DP), lambda i: (i, 0)),
            pl.BlockSpec((BLK, DP), lambda i: (i, 0)),
            pl.BlockSpec((BLK, N_RELS), lambda i: (i, 0)),
            pl.BlockSpec((N_RELS, R), lambda i: (0, 0)),
            pl.BlockSpec((R, DP, DP), lambda i: (0, 0, 0)),
            pl.BlockSpec((1, R), lambda i: (0, 0)),
            pl.BlockSpec((1, 1), lambda i: (0, 0)),
            pl.BlockSpec((1, 1), lambda i: (0, 0)),
        ],
        out_specs=pl.BlockSpec((BLK, 1), lambda i: (i, 0)),
        out_shape=jax.ShapeDtypeStruct((B, 1), jnp.float32),
    )(gl, gr, rel1h, rel_table, w_bf, bb, tm, to)
    return out.reshape(B)


# speedup vs baseline: 3.2329x; 2.2840x over previous
import functools

import jax
import jax.numpy as jnp
from jax import lax
from jax.experimental import pallas as pl
from jax.experimental.pallas import tpu as pltpu
from jax.experimental.pallas import tpu_sc as plsc

V = 100000
D = 300
DP = 384
R = 10
N_RELS = 40
B = 4096

NC, NS = 2, 16
NW = NC * NS
SIDE_PER_W = B // NW

BLK = 512
NBLK = B // BLK

PAD_RB = 2000
NPB = V // PAD_RB


def _pad_body(x_ref, o_ref):
    o_ref[:, :D] = x_ref[...]


def _pad_table(term_table):
    return pl.pallas_call(
        _pad_body,
        grid=(NPB,),
        in_specs=[pl.BlockSpec((PAD_RB, D), lambda i: (i, 0))],
        out_specs=pl.BlockSpec((PAD_RB, DP), lambda i: (i, 0)),
        out_shape=jax.ShapeDtypeStruct((V, DP), jnp.float32),
    )(term_table)


def _sc_gather(table, t_l, t_r):
    mesh = plsc.VectorSubcoreMesh(core_axis_name="c", subcore_axis_name="s")

    @functools.partial(
        pl.kernel,
        mesh=mesh,
        out_type=(
            jax.ShapeDtypeStruct((B, DP), jnp.float32),
            jax.ShapeDtypeStruct((B, DP), jnp.float32),
        ),
        scratch_types=[
            pltpu.VMEM((SIDE_PER_W,), jnp.int32),
            pltpu.VMEM((SIDE_PER_W, DP), jnp.float32),
            pltpu.SemaphoreType.DMA,
        ],
    )
    def gather_kernel(table_hbm, tl_hbm, tr_hbm, ol_hbm, or_hbm,
                      idx_v, rows_v, sem):
        wid = lax.axis_index("s") * NC + lax.axis_index("c")
        base = wid * SIDE_PER_W
        for i_hbm, o_hbm in ((tl_hbm, ol_hbm), (tr_hbm, or_hbm)):
            pltpu.sync_copy(i_hbm.at[pl.ds(base, SIDE_PER_W)], idx_v)
            pltpu.async_copy(table_hbm.at[idx_v], rows_v, sem).wait()
            pltpu.sync_copy(rows_v, o_hbm.at[pl.ds(base, SIDE_PER_W)])

    return gather_kernel(table, t_l, t_r)


def _tc_body(l_ref, r_ref, rel1h_ref, relt_ref, w_ref, bb_ref, tm_ref,
             to_ref, out_ref):
    rel_emb = jnp.dot(rel1h_ref[...], relt_ref[...],
                      preferred_element_type=jnp.float32)
    lane = lax.broadcasted_iota(jnp.int32, (BLK, DP), 1)
    l_sel = jnp.where(lane < D, l_ref[...], 0.0)
    r_sel = jnp.where(lane < D, r_ref[...], 0.0)
    p = jnp.zeros((BLK, DP), jnp.float32)
    w = w_ref[...]
    for k in range(R):
        a = (l_sel * rel_emb[:, k:k + 1]).astype(jnp.bfloat16)
        p = p + jnp.dot(a, w[k], preferred_element_type=jnp.float32)
    energy = (jnp.sum(p * r_sel, axis=1, keepdims=True)
              + jnp.sum(rel_emb * bb_ref[...], axis=1, keepdims=True))
    out_ref[...] = energy * tm_ref[0, 0] + to_ref[0, 0]


def kernel(rels, terms_L, terms_R, term_table, rel_table, bil_w, bil_b,
           truth_multiplier, truth_offset):
    tpad = _pad_table(term_table)
    gl, gr = _sc_gather(tpad, terms_L, terms_R)

    rel1h = (rels[:, None] == jnp.arange(N_RELS, dtype=jnp.int32)[None, :])
    rel1h = rel1h.astype(jnp.float32)
    w_bf = jnp.pad(bil_w, ((0, 0), (0, DP - D), (0, DP - D))).astype(jnp.bfloat16)
    bb = bil_b.reshape(1, R)
    tm = truth_multiplier.reshape(1, 1)
    to = truth_offset.reshape(1, 1)

    out = pl.pallas_call(
        _tc_body,
        grid=(NBLK,),
        in_specs=[
            pl.BlockSpec((BLK, DP), lambda i: (i, 0)),
            pl.BlockSpec((BLK, DP), lambda i: (i, 0)),
            pl.BlockSpec((BLK, N_RELS), lambda i: (i, 0)),
            pl.BlockSpec((N_RELS, R), lambda i: (0, 0)),
            pl.BlockSpec((R, DP, DP), lambda i: (0, 0, 0)),
            pl.BlockSpec((1, R), lambda i: (0, 0)),
            pl.BlockSpec((1, 1), lambda i: (0, 0)),
            pl.BlockSpec((1, 1), lambda i: (0, 0)),
        ],
        out_specs=pl.BlockSpec((BLK, 1), lambda i: (i, 0)),
        out_shape=jax.ShapeDtypeStruct((B, 1), jnp.float32),
    )(gl, gr, rel1h, rel_table, w_bf, bb, tm, to)
    return out.reshape(B)


# speedup vs baseline: 3.9392x; 1.2185x over previous
import functools

import jax
import jax.numpy as jnp
from jax import lax
from jax.experimental import pallas as pl
from jax.experimental.pallas import tpu as pltpu
from jax.experimental.pallas import tpu_sc as plsc

V = 100000
D = 300
DP = 384
R = 10
N_RELS = 40
B = 4096

NC, NS = 2, 16
NW = NC * NS
SIDE_PER_W = B // NW

BLK = 512
NBLK = B // BLK

PAD_RB = 512
NPB = -(-V // PAD_RB)


def _pad_body(xt_ref, o_ref):
    o_ref[:, :D] = xt_ref[...].T


def _pad_table(term_table_t):
    return pl.pallas_call(
        _pad_body,
        grid=(NPB,),
        in_specs=[pl.BlockSpec((D, PAD_RB), lambda i: (0, i))],
        out_specs=pl.BlockSpec((PAD_RB, DP), lambda i: (i, 0)),
        out_shape=jax.ShapeDtypeStruct((V, DP), jnp.float32),
    )(term_table_t)


def _sc_gather(table, t_l, t_r):
    mesh = plsc.VectorSubcoreMesh(core_axis_name="c", subcore_axis_name="s")

    @functools.partial(
        pl.kernel,
        mesh=mesh,
        out_type=(
            jax.ShapeDtypeStruct((B, DP), jnp.float32),
            jax.ShapeDtypeStruct((B, DP), jnp.float32),
        ),
        scratch_types=[
            pltpu.VMEM((SIDE_PER_W,), jnp.int32),
            pltpu.VMEM((SIDE_PER_W, DP), jnp.float32),
            pltpu.SemaphoreType.DMA,
        ],
    )
    def gather_kernel(table_hbm, tl_hbm, tr_hbm, ol_hbm, or_hbm,
                      idx_v, rows_v, sem):
        wid = lax.axis_index("s") * NC + lax.axis_index("c")
        base = wid * SIDE_PER_W
        for i_hbm, o_hbm in ((tl_hbm, ol_hbm), (tr_hbm, or_hbm)):
            pltpu.sync_copy(i_hbm.at[pl.ds(base, SIDE_PER_W)], idx_v)
            pltpu.async_copy(table_hbm.at[idx_v], rows_v, sem).wait()
            pltpu.sync_copy(rows_v, o_hbm.at[pl.ds(base, SIDE_PER_W)])

    return gather_kernel(table, t_l, t_r)


def _tc_body(l_ref, r_ref, rel1h_ref, relt_ref, w_ref, bb_ref, tm_ref,
             to_ref, out_ref):
    rel_emb = jnp.dot(rel1h_ref[...], relt_ref[...],
                      preferred_element_type=jnp.float32)
    lane = lax.broadcasted_iota(jnp.int32, (BLK, DP), 1)
    l_sel = jnp.where(lane < D, l_ref[...], 0.0)
    r_sel = jnp.where(lane < D, r_ref[...], 0.0)
    p = jnp.zeros((BLK, DP), jnp.float32)
    w = w_ref[...]
    for k in range(R):
        a = (l_sel * rel_emb[:, k:k + 1]).astype(jnp.bfloat16)
        p = p + jnp.dot(a, w[k], preferred_element_type=jnp.float32)
    energy = (jnp.sum(p * r_sel, axis=1, keepdims=True)
              + jnp.sum(rel_emb * bb_ref[...], axis=1, keepdims=True))
    out_ref[...] = energy * tm_ref[0, 0] + to_ref[0, 0]


def kernel(rels, terms_L, terms_R, term_table, rel_table, bil_w, bil_b,
           truth_multiplier, truth_offset):
    tpad = _pad_table(term_table.T)
    gl, gr = _sc_gather(tpad, terms_L, terms_R)

    rel1h = (rels[:, None] == jnp.arange(N_RELS, dtype=jnp.int32)[None, :])
    rel1h = rel1h.astype(jnp.float32)
    w_bf = jnp.pad(bil_w, ((0, 0), (0, DP - D), (0, DP - D))).astype(jnp.bfloat16)
    bb = bil_b.reshape(1, R)
    tm = truth_multiplier.reshape(1, 1)
    to = truth_offset.reshape(1, 1)

    out = pl.pallas_call(
        _tc_body,
        grid=(NBLK,),
        in_specs=[
            pl.BlockSpec((BLK, DP), lambda i: (i, 0)),
            pl.BlockSpec((BLK, DP), lambda i: (i, 0)),
            pl.BlockSpec((BLK, N_RELS), lambda i: (i, 0)),
            pl.BlockSpec((N_RELS, R), lambda i: (0, 0)),
            pl.BlockSpec((R, DP, DP), lambda i: (0, 0, 0)),
            pl.BlockSpec((1, R), lambda i: (0, 0)),
            pl.BlockSpec((1, 1), lambda i: (0, 0)),
            pl.BlockSpec((1, 1), lambda i: (0, 0)),
        ],
        out_specs=pl.BlockSpec((BLK, 1), lambda i: (i, 0)),
        out_shape=jax.ShapeDtypeStruct((B, 1), jnp.float32),
    )(gl, gr, rel1h, rel_table, w_bf, bb, tm, to)
    return out.reshape(B)


# speedup vs baseline: 3.9403x; 1.0003x over previous
import functools

import jax
import jax.numpy as jnp
from jax import lax
from jax.experimental import pallas as pl
from jax.experimental.pallas import tpu as pltpu
from jax.experimental.pallas import tpu_sc as plsc

V = 100000
D = 300
DP = 384
R = 10
N_RELS = 40
B = 4096

NC, NS = 2, 16
NW = NC * NS
SIDE_PER_W = B // NW

BLK = 512
NBLK = B // BLK

PAD_RB = 512
NPB = -(-V // PAD_RB)


def _pad_body(xt_ref, o_ref):
    o_ref[:, :D] = xt_ref[...].T


def _pad_table(term_table_t):
    return pl.pallas_call(
        _pad_body,
        grid=(NPB,),
        in_specs=[pl.BlockSpec((D, PAD_RB), lambda i: (0, i))],
        out_specs=pl.BlockSpec((PAD_RB, DP), lambda i: (i, 0)),
        out_shape=jax.ShapeDtypeStruct((V, DP), jnp.float32),
        compiler_params=pltpu.CompilerParams(
            dimension_semantics=("parallel",)),
    )(term_table_t)


def _sc_gather(table, t_l, t_r):
    mesh = plsc.VectorSubcoreMesh(core_axis_name="c", subcore_axis_name="s")

    @functools.partial(
        pl.kernel,
        mesh=mesh,
        out_type=(
            jax.ShapeDtypeStruct((B, DP), jnp.float32),
            jax.ShapeDtypeStruct((B, DP), jnp.float32),
        ),
        scratch_types=[
            pltpu.VMEM((SIDE_PER_W,), jnp.int32),
            pltpu.VMEM((SIDE_PER_W, DP), jnp.float32),
            pltpu.SemaphoreType.DMA,
        ],
    )
    def gather_kernel(table_hbm, tl_hbm, tr_hbm, ol_hbm, or_hbm,
                      idx_v, rows_v, sem):
        wid = lax.axis_index("s") * NC + lax.axis_index("c")
        base = wid * SIDE_PER_W
        for i_hbm, o_hbm in ((tl_hbm, ol_hbm), (tr_hbm, or_hbm)):
            pltpu.sync_copy(i_hbm.at[pl.ds(base, SIDE_PER_W)], idx_v)
            pltpu.async_copy(table_hbm.at[idx_v], rows_v, sem).wait()
            pltpu.sync_copy(rows_v, o_hbm.at[pl.ds(base, SIDE_PER_W)])

    return gather_kernel(table, t_l, t_r)


def _tc_body(l_ref, r_ref, rel1h_ref, relt_ref, w_ref, bb_ref, tm_ref,
             to_ref, out_ref):
    rel_emb = jnp.dot(rel1h_ref[...], relt_ref[...],
                      preferred_element_type=jnp.float32)
    lane = lax.broadcasted_iota(jnp.int32, (BLK, DP), 1)
    l_sel = jnp.where(lane < D, l_ref[...], 0.0)
    r_sel = jnp.where(lane < D, r_ref[...], 0.0)
    p = jnp.zeros((BLK, DP), jnp.float32)
    w = w_ref[...]
    for k in range(R):
        a = (l_sel * rel_emb[:, k:k + 1]).astype(jnp.bfloat16)
        p = p + jnp.dot(a, w[k], preferred_element_type=jnp.float32)
    energy = (jnp.sum(p * r_sel, axis=1, keepdims=True)
              + jnp.sum(rel_emb * bb_ref[...], axis=1, keepdims=True))
    out_ref[...] = energy * tm_ref[0, 0] + to_ref[0, 0]


def kernel(rels, terms_L, terms_R, term_table, rel_table, bil_w, bil_b,
           truth_multiplier, truth_offset):
    tpad = _pad_table(term_table.T)
    gl, gr = _sc_gather(tpad, terms_L, terms_R)

    rel1h = (rels[:, None] == jnp.arange(N_RELS, dtype=jnp.int32)[None, :])
    rel1h = rel1h.astype(jnp.float32)
    w_bf = jnp.pad(bil_w, ((0, 0), (0, DP - D), (0, DP - D))).astype(jnp.bfloat16)
    bb = bil_b.reshape(1, R)
    tm = truth_multiplier.reshape(1, 1)
    to = truth_offset.reshape(1, 1)

    out = pl.pallas_call(
        _tc_body,
        grid=(NBLK,),
        in_specs=[
            pl.BlockSpec((BLK, DP), lambda i: (i, 0)),
            pl.BlockSpec((BLK, DP), lambda i: (i, 0)),
            pl.BlockSpec((BLK, N_RELS), lambda i: (i, 0)),
            pl.BlockSpec((N_RELS, R), lambda i: (0, 0)),
            pl.BlockSpec((R, DP, DP), lambda i: (0, 0, 0)),
            pl.BlockSpec((1, R), lambda i: (0, 0)),
            pl.BlockSpec((1, 1), lambda i: (0, 0)),
            pl.BlockSpec((1, 1), lambda i: (0, 0)),
        ],
        out_specs=pl.BlockSpec((BLK, 1), lambda i: (i, 0)),
        out_shape=jax.ShapeDtypeStruct((B, 1), jnp.float32),
        compiler_params=pltpu.CompilerParams(
            dimension_semantics=("parallel",)),
    )(gl, gr, rel1h, rel_table, w_bf, bb, tm, to)
    return out.reshape(B)


# speedup vs baseline: 5.9492x; 1.5098x over previous
import functools

import jax
import jax.numpy as jnp
from jax import lax
from jax.experimental import pallas as pl
from jax.experimental.pallas import tpu as pltpu
from jax.experimental.pallas import tpu_sc as plsc

V = 100000
D = 300
DP = 384
R = 10
N_RELS = 40
B = 4096

NC, NS = 2, 16
NW = NC * NS
SIDE_PER_W = B // NW

BLK = 512
NBLK = B // BLK

PAD_RB = 2048
NPB = -(-V // PAD_RB)


def _pad_body(xt_ref, o_ref):
    o_ref[:, :D] = xt_ref[...].T


def _pad_table(term_table_t):
    return pl.pallas_call(
        _pad_body,
        grid=(NPB,),
        in_specs=[pl.BlockSpec((D, PAD_RB), lambda i: (0, i))],
        out_specs=pl.BlockSpec((PAD_RB, DP), lambda i: (i, 0)),
        out_shape=jax.ShapeDtypeStruct((V, DP), jnp.float32),
        compiler_params=pltpu.CompilerParams(
            dimension_semantics=("parallel",)),
    )(term_table_t)


def _sc_gather(table, t_l, t_r):
    mesh = plsc.VectorSubcoreMesh(core_axis_name="c", subcore_axis_name="s")

    @functools.partial(
        pl.kernel,
        mesh=mesh,
        out_type=(
            jax.ShapeDtypeStruct((B, DP), jnp.float32),
            jax.ShapeDtypeStruct((B, DP), jnp.float32),
        ),
        scratch_types=[
            pltpu.VMEM((SIDE_PER_W,), jnp.int32),
            pltpu.VMEM((SIDE_PER_W, DP), jnp.float32),
            pltpu.SemaphoreType.DMA,
        ],
    )
    def gather_kernel(table_hbm, tl_hbm, tr_hbm, ol_hbm, or_hbm,
                      idx_v, rows_v, sem):
        wid = lax.axis_index("s") * NC + lax.axis_index("c")
        base = wid * SIDE_PER_W
        for i_hbm, o_hbm in ((tl_hbm, ol_hbm), (tr_hbm, or_hbm)):
            pltpu.sync_copy(i_hbm.at[pl.ds(base, SIDE_PER_W)], idx_v)
            pltpu.async_copy(table_hbm.at[idx_v], rows_v, sem).wait()
            pltpu.sync_copy(rows_v, o_hbm.at[pl.ds(base, SIDE_PER_W)])

    return gather_kernel(table, t_l, t_r)


def _tc_body(l_ref, r_ref, rel1h_ref, relt_ref, w_ref, bb_ref, tm_ref,
             to_ref, out_ref):
    rel_emb = jnp.dot(rel1h_ref[...], relt_ref[...],
                      preferred_element_type=jnp.float32)
    lane = lax.broadcasted_iota(jnp.int32, (BLK, DP), 1)
    l_sel = jnp.where(lane < D, l_ref[...], 0.0)
    r_sel = jnp.where(lane < D, r_ref[...], 0.0)
    p = jnp.zeros((BLK, DP), jnp.float32)
    w = w_ref[...]
    for k in range(R):
        a = (l_sel * rel_emb[:, k:k + 1]).astype(jnp.bfloat16)
        p = p + jnp.dot(a, w[k], preferred_element_type=jnp.float32)
    energy = (jnp.sum(p * r_sel, axis=1, keepdims=True)
              + jnp.sum(rel_emb * bb_ref[...], axis=1, keepdims=True))
    out_ref[...] = energy * tm_ref[0, 0] + to_ref[0, 0]


def kernel(rels, terms_L, terms_R, term_table, rel_table, bil_w, bil_b,
           truth_multiplier, truth_offset):
    tpad = _pad_table(term_table.T)
    gl, gr = _sc_gather(tpad, terms_L, terms_R)

    rel1h = (rels[:, None] == jnp.arange(N_RELS, dtype=jnp.int32)[None, :])
    rel1h = rel1h.astype(jnp.float32)
    w_bf = jnp.pad(bil_w, ((0, 0), (0, DP - D), (0, DP - D))).astype(jnp.bfloat16)
    bb = bil_b.reshape(1, R)
    tm = truth_multiplier.reshape(1, 1)
    to = truth_offset.reshape(1, 1)

    out = pl.pallas_call(
        _tc_body,
        grid=(NBLK,),
        in_specs=[
            pl.BlockSpec((BLK, DP), lambda i: (i, 0)),
            pl.BlockSpec((BLK, DP), lambda i: (i, 0)),
            pl.BlockSpec((BLK, N_RELS), lambda i: (i, 0)),
            pl.BlockSpec((N_RELS, R), lambda i: (0, 0)),
            pl.BlockSpec((R, DP, DP), lambda i: (0, 0, 0)),
            pl.BlockSpec((1, R), lambda i: (0, 0)),
            pl.BlockSpec((1, 1), lambda i: (0, 0)),
            pl.BlockSpec((1, 1), lambda i: (0, 0)),
        ],
        out_specs=pl.BlockSpec((BLK, 1), lambda i: (i, 0)),
        out_shape=jax.ShapeDtypeStruct((B, 1), jnp.float32),
        compiler_params=pltpu.CompilerParams(
            dimension_semantics=("parallel",)),
    )(gl, gr, rel1h, rel_table, w_bf, bb, tm, to)
    return out.reshape(B)


# speedup vs baseline: 6.2326x; 1.0476x over previous
import functools

import jax
import jax.numpy as jnp
from jax import lax
from jax.experimental import pallas as pl
from jax.experimental.pallas import tpu as pltpu
from jax.experimental.pallas import tpu_sc as plsc

V = 100000
D = 300
DP = 384
R = 10
N_RELS = 40
B = 4096

NC, NS = 2, 16
NW = NC * NS
SIDE_PER_W = B // NW

BLK = 512
NBLK = B // BLK

PAD_RB = 4096
NPB = -(-V // PAD_RB)


def _pad_body(xt_ref, o_ref):
    o_ref[:, :D] = xt_ref[...].T


def _pad_table(term_table_t):
    return pl.pallas_call(
        _pad_body,
        grid=(NPB,),
        in_specs=[pl.BlockSpec((D, PAD_RB), lambda i: (0, i))],
        out_specs=pl.BlockSpec((PAD_RB, DP), lambda i: (i, 0)),
        out_shape=jax.ShapeDtypeStruct((V, DP), jnp.float32),
        compiler_params=pltpu.CompilerParams(
            dimension_semantics=("parallel",)),
    )(term_table_t)


def _sc_gather(table, t_l, t_r):
    mesh = plsc.VectorSubcoreMesh(core_axis_name="c", subcore_axis_name="s")

    @functools.partial(
        pl.kernel,
        mesh=mesh,
        out_type=(
            jax.ShapeDtypeStruct((B, DP), jnp.float32),
            jax.ShapeDtypeStruct((B, DP), jnp.float32),
        ),
        scratch_types=[
            pltpu.VMEM((SIDE_PER_W,), jnp.int32),
            pltpu.VMEM((SIDE_PER_W, DP), jnp.float32),
            pltpu.SemaphoreType.DMA,
        ],
    )
    def gather_kernel(table_hbm, tl_hbm, tr_hbm, ol_hbm, or_hbm,
                      idx_v, rows_v, sem):
        wid = lax.axis_index("s") * NC + lax.axis_index("c")
        base = wid * SIDE_PER_W
        for i_hbm, o_hbm in ((tl_hbm, ol_hbm), (tr_hbm, or_hbm)):
            pltpu.sync_copy(i_hbm.at[pl.ds(base, SIDE_PER_W)], idx_v)
            pltpu.async_copy(table_hbm.at[idx_v], rows_v, sem).wait()
            pltpu.sync_copy(rows_v, o_hbm.at[pl.ds(base, SIDE_PER_W)])

    return gather_kernel(table, t_l, t_r)


def _tc_body(l_ref, r_ref, rel1h_ref, relt_ref, w_ref, bb_ref, tm_ref,
             to_ref, out_ref):
    rel_emb = jnp.dot(rel1h_ref[...], relt_ref[...],
                      preferred_element_type=jnp.float32)
    lane = lax.broadcasted_iota(jnp.int32, (BLK, DP), 1)
    l_sel = jnp.where(lane < D, l_ref[...], 0.0)
    r_sel = jnp.where(lane < D, r_ref[...], 0.0)
    p = jnp.zeros((BLK, DP), jnp.float32)
    w = w_ref[...]
    for k in range(R):
        a = (l_sel * rel_emb[:, k:k + 1]).astype(jnp.bfloat16)
        p = p + jnp.dot(a, w[k], preferred_element_type=jnp.float32)
    energy = (jnp.sum(p * r_sel, axis=1, keepdims=True)
              + jnp.sum(rel_emb * bb_ref[...], axis=1, keepdims=True))
    out_ref[...] = energy * tm_ref[0, 0] + to_ref[0, 0]


def kernel(rels, terms_L, terms_R, term_table, rel_table, bil_w, bil_b,
           truth_multiplier, truth_offset):
    tpad = _pad_table(term_table.T)
    gl, gr = _sc_gather(tpad, terms_L, terms_R)

    rel1h = (rels[:, None] == jnp.arange(N_RELS, dtype=jnp.int32)[None, :])
    rel1h = rel1h.astype(jnp.float32)
    w_bf = jnp.pad(bil_w, ((0, 0), (0, DP - D), (0, DP - D))).astype(jnp.bfloat16)
    bb = bil_b.reshape(1, R)
    tm = truth_multiplier.reshape(1, 1)
    to = truth_offset.reshape(1, 1)

    out = pl.pallas_call(
        _tc_body,
        grid=(NBLK,),
        in_specs=[
            pl.BlockSpec((BLK, DP), lambda i: (i, 0)),
            pl.BlockSpec((BLK, DP), lambda i: (i, 0)),
            pl.BlockSpec((BLK, N_RELS), lambda i: (i, 0)),
            pl.BlockSpec((N_RELS, R), lambda i: (0, 0)),
            pl.BlockSpec((R, DP, DP), lambda i: (0, 0, 0)),
            pl.BlockSpec((1, R), lambda i: (0, 0)),
            pl.BlockSpec((1, 1), lambda i: (0, 0)),
            pl.BlockSpec((1, 1), lambda i: (0, 0)),
        ],
        out_specs=pl.BlockSpec((BLK, 1), lambda i: (i, 0)),
        out_shape=jax.ShapeDtypeStruct((B, 1), jnp.float32),
        compiler_params=pltpu.CompilerParams(
            dimension_semantics=("parallel",)),
    )(gl, gr, rel1h, rel_table, w_bf, bb, tm, to)
    return out.reshape(B)


# speedup vs baseline: 7.2116x; 1.1571x over previous
import functools

import jax
import jax.numpy as jnp
from jax import lax
from jax.experimental import pallas as pl
from jax.experimental.pallas import tpu as pltpu
from jax.experimental.pallas import tpu_sc as plsc

V = 100000
D = 300
DPK = 256
DW = 384
R = 10
N_RELS = 40
B = 4096

NC, NS = 2, 16
NW = NC * NS
SIDE_PER_W = B // NW

BLK = 512
NBLK = B // BLK

PAD_RB = 4096
NPB = -(-V // PAD_RB)


def _pad_body(xt_ref, o_ref):
    xt = xt_ref[...].T.astype(jnp.bfloat16)
    z = lax.bitcast_convert_type(xt, jnp.uint16).astype(jnp.uint32)
    zw = jnp.pad(z, ((0, 0), (0, 2 * DPK - D)))
    hi = pltpu.roll(zw, DPK, 1)[:, :DPK]
    packed = zw[:, :DPK] | (hi << 16)
    o_ref[...] = lax.bitcast_convert_type(packed, jnp.int32)


def _pad_table(term_table_t):
    return pl.pallas_call(
        _pad_body,
        grid=(NPB,),
        in_specs=[pl.BlockSpec((D, PAD_RB), lambda i: (0, i))],
        out_specs=pl.BlockSpec((PAD_RB, DPK), lambda i: (i, 0)),
        out_shape=jax.ShapeDtypeStruct((V, DPK), jnp.int32),
        compiler_params=pltpu.CompilerParams(
            dimension_semantics=("parallel",)),
    )(term_table_t)


def _sc_gather(table, t_l, t_r):
    mesh = plsc.VectorSubcoreMesh(core_axis_name="c", subcore_axis_name="s")

    @functools.partial(
        pl.kernel,
        mesh=mesh,
        out_type=(
            jax.ShapeDtypeStruct((B, DPK), jnp.int32),
            jax.ShapeDtypeStruct((B, DPK), jnp.int32),
        ),
        scratch_types=[
            pltpu.VMEM((SIDE_PER_W,), jnp.int32),
            pltpu.VMEM((SIDE_PER_W, DPK), jnp.int32),
            pltpu.SemaphoreType.DMA,
        ],
    )
    def gather_kernel(table_hbm, tl_hbm, tr_hbm, ol_hbm, or_hbm,
                      idx_v, rows_v, sem):
        wid = lax.axis_index("s") * NC + lax.axis_index("c")
        base = wid * SIDE_PER_W
        for i_hbm, o_hbm in ((tl_hbm, ol_hbm), (tr_hbm, or_hbm)):
            pltpu.sync_copy(i_hbm.at[pl.ds(base, SIDE_PER_W)], idx_v)
            pltpu.async_copy(table_hbm.at[idx_v], rows_v, sem).wait()
            pltpu.sync_copy(rows_v, o_hbm.at[pl.ds(base, SIDE_PER_W)])

    return gather_kernel(table, t_l, t_r)


def _unpack(v):
    lo = lax.bitcast_convert_type(v << 16, jnp.float32)
    hi = lax.bitcast_convert_type(
        v & jnp.int32(-65536), jnp.float32)
    return jnp.concatenate([lo, hi[:, :DW - DPK]], axis=1)


def _tc_body(l_ref, r_ref, rel1h_ref, relt_ref, w_ref, bb_ref, tm_ref,
             to_ref, out_ref):
    rel_emb = jnp.dot(rel1h_ref[...], relt_ref[...],
                      preferred_element_type=jnp.float32)
    l32 = _unpack(l_ref[...])
    r_sel = _unpack(r_ref[...])
    p = jnp.zeros((BLK, DW), jnp.float32)
    w = w_ref[...]
    for k in range(R):
        a = (l32 * rel_emb[:, k:k + 1]).astype(jnp.bfloat16)
        p = p + jnp.dot(a, w[k], preferred_element_type=jnp.float32)
    energy = (jnp.sum(p * r_sel, axis=1, keepdims=True)
              + jnp.sum(rel_emb * bb_ref[...], axis=1, keepdims=True))
    out_ref[...] = energy * tm_ref[0, 0] + to_ref[0, 0]


def kernel(rels, terms_L, terms_R, term_table, rel_table, bil_w, bil_b,
           truth_multiplier, truth_offset):
    tpad = _pad_table(term_table.T)
    gl, gr = _sc_gather(tpad, terms_L, terms_R)

    rel1h = (rels[:, None] == jnp.arange(N_RELS, dtype=jnp.int32)[None, :])
    rel1h = rel1h.astype(jnp.float32)
    w_bf = jnp.pad(bil_w, ((0, 0), (0, DW - D), (0, DW - D))).astype(jnp.bfloat16)
    bb = bil_b.reshape(1, R)
    tm = truth_multiplier.reshape(1, 1)
    to = truth_offset.reshape(1, 1)

    out = pl.pallas_call(
        _tc_body,
        grid=(NBLK,),
        in_specs=[
            pl.BlockSpec((BLK, DPK), lambda i: (i, 0)),
            pl.BlockSpec((BLK, DPK), lambda i: (i, 0)),
            pl.BlockSpec((BLK, N_RELS), lambda i: (i, 0)),
            pl.BlockSpec((N_RELS, R), lambda i: (0, 0)),
            pl.BlockSpec((R, DW, DW), lambda i: (0, 0, 0)),
            pl.BlockSpec((1, R), lambda i: (0, 0)),
            pl.BlockSpec((1, 1), lambda i: (0, 0)),
            pl.BlockSpec((1, 1), lambda i: (0, 0)),
        ],
        out_specs=pl.BlockSpec((BLK, 1), lambda i: (i, 0)),
        out_shape=jax.ShapeDtypeStruct((B, 1), jnp.float32),
        compiler_params=pltpu.CompilerParams(
            dimension_semantics=("parallel",)),
    )(gl, gr, rel1h, rel_table, w_bf, bb, tm, to)
    return out.reshape(B)


# speedup vs baseline: 7.2773x; 1.0091x over previous
import functools

import jax
import jax.numpy as jnp
from jax import lax
from jax.experimental import pallas as pl
from jax.experimental.pallas import tpu as pltpu
from jax.experimental.pallas import tpu_sc as plsc

V = 100000
D = 300
DPK = 256
DW = 384
R = 10
N_RELS = 40
B = 4096

NC, NS = 2, 16
NW = NC * NS
SIDE_PER_W = B // NW

BLK = 1024
NBLK = B // BLK

PAD_RB = 4096
NPB = -(-V // PAD_RB)


def _pad_body(xt_ref, o_ref):
    xt = xt_ref[...].T.astype(jnp.bfloat16)
    z = lax.bitcast_convert_type(xt, jnp.uint16).astype(jnp.uint32)
    zw = jnp.pad(z, ((0, 0), (0, 2 * DPK - D)))
    hi = pltpu.roll(zw, DPK, 1)[:, :DPK]
    packed = zw[:, :DPK] | (hi << 16)
    o_ref[...] = lax.bitcast_convert_type(packed, jnp.int32)


def _pad_table(term_table_t):
    return pl.pallas_call(
        _pad_body,
        grid=(NPB,),
        in_specs=[pl.BlockSpec((D, PAD_RB), lambda i: (0, i))],
        out_specs=pl.BlockSpec((PAD_RB, DPK), lambda i: (i, 0)),
        out_shape=jax.ShapeDtypeStruct((V, DPK), jnp.int32),
        compiler_params=pltpu.CompilerParams(
            dimension_semantics=("parallel",)),
    )(term_table_t)


def _sc_gather(table, t_l, t_r):
    mesh = plsc.VectorSubcoreMesh(core_axis_name="c", subcore_axis_name="s")

    @functools.partial(
        pl.kernel,
        mesh=mesh,
        out_type=(
            jax.ShapeDtypeStruct((B, DPK), jnp.int32),
            jax.ShapeDtypeStruct((B, DPK), jnp.int32),
        ),
        scratch_types=[
            pltpu.VMEM((SIDE_PER_W,), jnp.int32),
            pltpu.VMEM((SIDE_PER_W, DPK), jnp.int32),
            pltpu.SemaphoreType.DMA,
        ],
    )
    def gather_kernel(table_hbm, tl_hbm, tr_hbm, ol_hbm, or_hbm,
                      idx_v, rows_v, sem):
        wid = lax.axis_index("s") * NC + lax.axis_index("c")
        base = wid * SIDE_PER_W
        for i_hbm, o_hbm in ((tl_hbm, ol_hbm), (tr_hbm, or_hbm)):
            pltpu.sync_copy(i_hbm.at[pl.ds(base, SIDE_PER_W)], idx_v)
            pltpu.async_copy(table_hbm.at[idx_v], rows_v, sem).wait()
            pltpu.sync_copy(rows_v, o_hbm.at[pl.ds(base, SIDE_PER_W)])

    return gather_kernel(table, t_l, t_r)


def _unpack(v):
    lo = lax.bitcast_convert_type(v << 16, jnp.float32)
    hi = lax.bitcast_convert_type(
        v & jnp.int32(-65536), jnp.float32)
    return jnp.concatenate([lo, hi[:, :DW - DPK]], axis=1)


def _tc_body(l_ref, r_ref, rel1h_ref, relt_ref, w_ref, bb_ref, tm_ref,
             to_ref, out_ref):
    rel_emb = jnp.dot(rel1h_ref[...], relt_ref[...],
                      preferred_element_type=jnp.float32)
    l32 = _unpack(l_ref[...])
    r_sel = _unpack(r_ref[...])
    p = jnp.zeros((BLK, DW), jnp.float32)
    w = w_ref[...]
    for k in range(R):
        a = (l32 * rel_emb[:, k:k + 1]).astype(jnp.bfloat16)
        p = p + jnp.dot(a, w[k], preferred_element_type=jnp.float32)
    energy = (jnp.sum(p * r_sel, axis=1, keepdims=True)
              + jnp.sum(rel_emb * bb_ref[...], axis=1, keepdims=True))
    out_ref[...] = energy * tm_ref[0, 0] + to_ref[0, 0]


def kernel(rels, terms_L, terms_R, term_table, rel_table, bil_w, bil_b,
           truth_multiplier, truth_offset):
    tpad = _pad_table(term_table.T)
    gl, gr = _sc_gather(tpad, terms_L, terms_R)

    rel1h = (rels[:, None] == jnp.arange(N_RELS, dtype=jnp.int32)[None, :])
    rel1h = rel1h.astype(jnp.float32)
    w_bf = jnp.pad(bil_w.astype(jnp.bfloat16),
                   ((0, 0), (0, DW - D), (0, DW - D)))
    bb = bil_b.reshape(1, R)
    tm = truth_multiplier.reshape(1, 1)
    to = truth_offset.reshape(1, 1)

    out = pl.pallas_call(
        _tc_body,
        grid=(NBLK,),
        in_specs=[
            pl.BlockSpec((BLK, DPK), lambda i: (i, 0)),
            pl.BlockSpec((BLK, DPK), lambda i: (i, 0)),
            pl.BlockSpec((BLK, N_RELS), lambda i: (i, 0)),
            pl.BlockSpec((N_RELS, R), lambda i: (0, 0)),
            pl.BlockSpec((R, DW, DW), lambda i: (0, 0, 0)),
            pl.BlockSpec((1, R), lambda i: (0, 0)),
            pl.BlockSpec((1, 1), lambda i: (0, 0)),
            pl.BlockSpec((1, 1), lambda i: (0, 0)),
        ],
        out_specs=pl.BlockSpec((BLK, 1), lambda i: (i, 0)),
        out_shape=jax.ShapeDtypeStruct((B, 1), jnp.float32),
        compiler_params=pltpu.CompilerParams(
            dimension_semantics=("parallel",)),
    )(gl, gr, rel1h, rel_table, w_bf, bb, tm, to)
    return out.reshape(B)


# speedup vs baseline: 7.3622x; 1.0117x over previous
import functools

import jax
import jax.numpy as jnp
from jax import lax
from jax.experimental import pallas as pl
from jax.experimental.pallas import tpu as pltpu
from jax.experimental.pallas import tpu_sc as plsc

V = 100000
D = 300
DPK = 256
DW = 384
R = 10
N_RELS = 40
B = 4096

NC, NS = 2, 16
NW = NC * NS
SIDE_PER_W = B // NW

BLK = 1024
NBLK = B // BLK

PAD_RB = 8192
NPB = -(-V // PAD_RB)


def _pad_body(xt_ref, o_ref):
    xt = xt_ref[...].T.astype(jnp.bfloat16)
    z = lax.bitcast_convert_type(xt, jnp.uint16).astype(jnp.uint32)
    zw = jnp.pad(z, ((0, 0), (0, 2 * DPK - D)))
    hi = pltpu.roll(zw, DPK, 1)[:, :DPK]
    packed = zw[:, :DPK] | (hi << 16)
    o_ref[...] = lax.bitcast_convert_type(packed, jnp.int32)


def _pad_table(term_table_t):
    return pl.pallas_call(
        _pad_body,
        grid=(NPB,),
        in_specs=[pl.BlockSpec((D, PAD_RB), lambda i: (0, i))],
        out_specs=pl.BlockSpec((PAD_RB, DPK), lambda i: (i, 0)),
        out_shape=jax.ShapeDtypeStruct((V, DPK), jnp.int32),
        compiler_params=pltpu.CompilerParams(
            dimension_semantics=("parallel",)),
    )(term_table_t)


def _sc_gather(table, t_l, t_r):
    mesh = plsc.VectorSubcoreMesh(core_axis_name="c", subcore_axis_name="s")

    @functools.partial(
        pl.kernel,
        mesh=mesh,
        out_type=(
            jax.ShapeDtypeStruct((B, DPK), jnp.int32),
            jax.ShapeDtypeStruct((B, DPK), jnp.int32),
        ),
        scratch_types=[
            pltpu.VMEM((SIDE_PER_W,), jnp.int32),
            pltpu.VMEM((SIDE_PER_W, DPK), jnp.int32),
            pltpu.SemaphoreType.DMA,
        ],
    )
    def gather_kernel(table_hbm, tl_hbm, tr_hbm, ol_hbm, or_hbm,
                      idx_v, rows_v, sem):
        wid = lax.axis_index("s") * NC + lax.axis_index("c")
        base = wid * SIDE_PER_W
        for i_hbm, o_hbm in ((tl_hbm, ol_hbm), (tr_hbm, or_hbm)):
            pltpu.sync_copy(i_hbm.at[pl.ds(base, SIDE_PER_W)], idx_v)
            pltpu.async_copy(table_hbm.at[idx_v], rows_v, sem).wait()
            pltpu.sync_copy(rows_v, o_hbm.at[pl.ds(base, SIDE_PER_W)])

    return gather_kernel(table, t_l, t_r)


def _unpack(v):
    lo = lax.bitcast_convert_type(v << 16, jnp.float32)
    hi = lax.bitcast_convert_type(
        v & jnp.int32(-65536), jnp.float32)
    return jnp.concatenate([lo, hi[:, :DW - DPK]], axis=1)


def _tc_body(l_ref, r_ref, rel1h_ref, relt_ref, w_ref, bb_ref, tm_ref,
             to_ref, out_ref):
    rel_emb = jnp.dot(rel1h_ref[...], relt_ref[...],
                      preferred_element_type=jnp.float32)
    l32 = _unpack(l_ref[...])
    r_sel = _unpack(r_ref[...])
    p = jnp.zeros((BLK, DW), jnp.float32)
    w = w_ref[...]
    for k in range(R):
        a = (l32 * rel_emb[:, k:k + 1]).astype(jnp.bfloat16)
        p = p + jnp.dot(a, w[k], preferred_element_type=jnp.float32)
    energy = (jnp.sum(p * r_sel, axis=1, keepdims=True)
              + jnp.sum(rel_emb * bb_ref[...], axis=1, keepdims=True))
    out_ref[...] = energy * tm_ref[0, 0] + to_ref[0, 0]


def kernel(rels, terms_L, terms_R, term_table, rel_table, bil_w, bil_b,
           truth_multiplier, truth_offset):
    tpad = _pad_table(term_table.T)
    gl, gr = _sc_gather(tpad, terms_L, terms_R)

    rel1h = (rels[:, None] == jnp.arange(N_RELS, dtype=jnp.int32)[None, :])
    rel1h = rel1h.astype(jnp.float32)
    w_bf = jnp.pad(bil_w.astype(jnp.bfloat16),
                   ((0, 0), (0, DW - D), (0, DW - D)))
    bb = bil_b.reshape(1, R)
    tm = truth_multiplier.reshape(1, 1)
    to = truth_offset.reshape(1, 1)

    out = pl.pallas_call(
        _tc_body,
        grid=(NBLK,),
        in_specs=[
            pl.BlockSpec((BLK, DPK), lambda i: (i, 0)),
            pl.BlockSpec((BLK, DPK), lambda i: (i, 0)),
            pl.BlockSpec((BLK, N_RELS), lambda i: (i, 0)),
            pl.BlockSpec((N_RELS, R), lambda i: (0, 0)),
            pl.BlockSpec((R, DW, DW), lambda i: (0, 0, 0)),
            pl.BlockSpec((1, R), lambda i: (0, 0)),
            pl.BlockSpec((1, 1), lambda i: (0, 0)),
            pl.BlockSpec((1, 1), lambda i: (0, 0)),
        ],
        out_specs=pl.BlockSpec((BLK, 1), lambda i: (i, 0)),
        out_shape=jax.ShapeDtypeStruct((B, 1), jnp.float32),
        compiler_params=pltpu.CompilerParams(
            dimension_semantics=("parallel",)),
    )(gl, gr, rel1h, rel_table, w_bf, bb, tm, to)
    return out.reshape(B)
